# baseline (speedup 1.0000x reference)
import functools

import jax
import jax.numpy as jnp
from jax.experimental import pallas as pl
from jax.experimental.pallas import tpu as pltpu

_B = 16
_N = 4096
_T = 1024


def _body(ui_ref, c3_ref, c2_ref, c1_ref,
          W1_ref, b1_ref, W2_ref, b2_ref, Wr_ref, br_ref, Wt_ref, bt_ref,
          Wg1_ref, bg1_ref, Wg2_ref, bg2_ref, Wgs_ref, bgs_ref,
          Wo1_ref, bo1_ref, Wo2_ref, bo2_ref, Wo3_ref, bo3_ref,
          Wa1_ref, ba1_ref, Wa2_ref, ba2_ref, Wa3_ref, ba3_ref,
          wts_ref, sdf_ref, img_ref, aud_ref,
          tbg_ref, tbo_ref, tba_ref):
    b = pl.program_id(0)
    n = pl.program_id(1)

    def dot(x, y):
        return jnp.dot(x, y, preferred_element_type=jnp.float32)

    @pl.when((b == 0) & (n == 0))
    def _router():
        hs = ui_ref[...]
        for _ in range(3):
            m = jnp.tanh(dot(hs, W1_ref[...]) + b1_ref[...])
            hs = hs + jnp.tanh(dot(m, W2_ref[...]) + b2_ref[...])
        logits = dot(hs, Wr_ref[...]) + br_ref[...]
        wts_ref[...] = jax.nn.softmax(logits, axis=-1)
        thought = jnp.tanh(dot(hs, Wt_ref[...]) + bt_ref[...])
        tbg_ref[...] = dot(thought, Wg1_ref[3:, :]) + bg1_ref[...]
        tbo_ref[...] = dot(thought, Wo1_ref[2:, :]) + bo1_ref[...]
        tba_ref[...] = dot(thought, Wa1_ref[1:, :]) + ba1_ref[...]

    w_row = wts_ref[pl.ds(b, 1), :]

    def layer1(c_ref, W1r, tb_ref):
        c = c_ref[0]
        acc = tb_ref[pl.ds(b, 1), :]
        for i in range(c.shape[-1]):
            acc = acc + c[:, i:i + 1] * W1r[i:i + 1, :]
        return jnp.maximum(acc, 0.0)

    h = layer1(c3_ref, Wg1_ref, tbg_ref)
    h = jnp.maximum(dot(h, Wg2_ref[...]) + bg2_ref[...], 0.0)
    sdf = jnp.sum(h * Wgs_ref[...], axis=-1, keepdims=True) + bgs_ref[...]
    sdf_ref[0] = sdf * w_row[:, 0:1]

    h = layer1(c2_ref, Wo1_ref, tbo_ref)
    h = jnp.maximum(dot(h, Wo2_ref[...]) + bo2_ref[...], 0.0)
    img = jnp.concatenate(
        [jnp.sum(h * Wo3_ref[i:i + 1, :], axis=-1, keepdims=True)
         for i in range(3)], axis=-1)
    img_ref[0] = jax.nn.sigmoid(img + bo3_ref[...]) * w_row[:, 1:2]

    h = layer1(c1_ref, Wa1_ref, tba_ref)
    h = jnp.maximum(dot(h, Wa2_ref[...]) + ba2_ref[...], 0.0)
    aud = jnp.tanh(jnp.sum(h * Wa3_ref[...], axis=-1, keepdims=True)
                   + ba3_ref[...])
    aud_ref[0] = aud * w_row[:, 2:3]


def _full(shape):
    return pl.BlockSpec(shape, lambda b, n: (0,) * len(shape))


@jax.jit
def kernel(user_intent, coords_3d, coords_2d, coords_1d, W1, b1, W2, b2, Wr,
           br, Wt, bt, Wg1, bg1, Wg2, bg2, Wgs, bgs, Wgc, bgc, Wo1, bo1, Wo2,
           bo2, Wo3, bo3, Wa1, ba1, Wa2, ba2, Wa3, ba3):
    del Wgc, bgc
    B, N, T = _B, _N, _T
    nt = N // T
    row = lambda v: v.reshape(1, -1)

    in_specs = [
        _full((B, 64)),
        pl.BlockSpec((1, T, 3), lambda b, n: (b, n, 0)),
        pl.BlockSpec((1, T, 2), lambda b, n: (b, n, 0)),
        pl.BlockSpec((1, T, 1), lambda b, n: (b, n, 0)),
        _full((64, 64)), _full((1, 64)), _full((64, 64)), _full((1, 64)),
        _full((64, 3)), _full((1, 3)), _full((64, 16)), _full((1, 16)),
        _full((19, 256)), _full((1, 256)), _full((256, 256)), _full((1, 256)),
        _full((1, 256)), _full((1, 1)),
        _full((18, 256)), _full((1, 256)), _full((256, 256)), _full((1, 256)),
        _full((3, 256)), _full((1, 3)),
        _full((17, 256)), _full((1, 256)), _full((256, 256)), _full((1, 256)),
        _full((1, 256)), _full((1, 1)),
    ]
    out_specs = [
        _full((B, 3)),
        pl.BlockSpec((1, T, 1), lambda b, n: (b, n, 0)),
        pl.BlockSpec((1, T, 3), lambda b, n: (b, n, 0)),
        pl.BlockSpec((1, T, 1), lambda b, n: (b, n, 0)),
    ]
    out_shapes = [
        jax.ShapeDtypeStruct((B, 3), jnp.float32),
        jax.ShapeDtypeStruct((B, N, 1), jnp.float32),
        jax.ShapeDtypeStruct((B, N, 3), jnp.float32),
        jax.ShapeDtypeStruct((B, N, 1), jnp.float32),
    ]

    weights, raw_sdf, raw_img, raw_audio = pl.pallas_call(
        _body,
        grid=(B, nt),
        in_specs=in_specs,
        out_specs=out_specs,
        out_shape=out_shapes,
        scratch_shapes=[
            pltpu.VMEM((B, 256), jnp.float32),
            pltpu.VMEM((B, 256), jnp.float32),
            pltpu.VMEM((B, 256), jnp.float32),
        ],
    )(user_intent, coords_3d, coords_2d, coords_1d,
      W1, row(b1), W2, row(b2), Wr, row(br), Wt, row(bt),
      Wg1, row(bg1), Wg2, row(bg2), Wgs.T, row(bgs),
      Wo1, row(bo1), Wo2, row(bo2), Wo3.T, row(bo3),
      Wa1, row(ba1), Wa2, row(ba2), Wa3.T, row(ba3))
    return weights, raw_sdf, raw_img, raw_audio


# speedup vs baseline: 1.4148x; 1.4148x over previous
import functools

import jax
import jax.numpy as jnp
from jax.experimental import pallas as pl
from jax.experimental.pallas import tpu as pltpu

_B = 16
_N = 4096
_T = 4096


def _body(ui_ref, c3_ref, c2_ref, c1_ref,
          W1_ref, b1_ref, W2_ref, b2_ref, Wr_ref, br_ref, Wt_ref, bt_ref,
          Wg1_ref, bg1_ref, Wg2_ref, bg2_ref, Wgs_ref, bgs_ref,
          Wo1_ref, bo1_ref, Wo2_ref, bo2_ref, Wo3_ref, bo3_ref,
          Wa1_ref, ba1_ref, Wa2_ref, ba2_ref, Wa3_ref, ba3_ref,
          wts_ref, sdf_ref, img_ref, aud_ref,
          tbg_ref, tbo_ref, tba_ref):
    b = pl.program_id(0)
    n = pl.program_id(1)

    def dot(x, y):
        return jnp.dot(x, y, preferred_element_type=jnp.float32)

    @pl.when((b == 0) & (n == 0))
    def _router():
        hs = ui_ref[...]
        for _ in range(3):
            m = jnp.tanh(dot(hs, W1_ref[...]) + b1_ref[...])
            hs = hs + jnp.tanh(dot(m, W2_ref[...]) + b2_ref[...])
        logits = dot(hs, Wr_ref[...]) + br_ref[...]
        wts_ref[...] = jax.nn.softmax(logits, axis=-1)
        thought = jnp.tanh(dot(hs, Wt_ref[...]) + bt_ref[...])
        tbg_ref[...] = dot(thought, Wg1_ref[3:, :]) + bg1_ref[...]
        tbo_ref[...] = dot(thought, Wo1_ref[2:, :]) + bo1_ref[...]
        tba_ref[...] = dot(thought, Wa1_ref[1:, :]) + ba1_ref[...]

    w_row = wts_ref[pl.ds(b, 1), :]

    h = jnp.maximum(dot(c3_ref[0], Wg1_ref[0:3, :]) + tbg_ref[pl.ds(b, 1), :], 0.0)
    h = jnp.maximum(dot(h, Wg2_ref[...]) + bg2_ref[...], 0.0)
    sdf = dot(h, Wgs_ref[...]) + bgs_ref[...]
    sdf_ref[0] = sdf * w_row[:, 0:1]

    h = jnp.maximum(dot(c2_ref[0], Wo1_ref[0:2, :]) + tbo_ref[pl.ds(b, 1), :], 0.0)
    h = jnp.maximum(dot(h, Wo2_ref[...]) + bo2_ref[...], 0.0)
    img = dot(h, Wo3_ref[...]) + bo3_ref[...]
    img_ref[0] = jax.nn.sigmoid(img) * w_row[:, 1:2]

    h = jnp.maximum(c1_ref[0] * Wa1_ref[0:1, :] + tba_ref[pl.ds(b, 1), :], 0.0)
    h = jnp.maximum(dot(h, Wa2_ref[...]) + ba2_ref[...], 0.0)
    aud = jnp.tanh(dot(h, Wa3_ref[...]) + ba3_ref[...])
    aud_ref[0] = aud * w_row[:, 2:3]


def _full(shape):
    return pl.BlockSpec(shape, lambda b, n: (0,) * len(shape))


@jax.jit
def kernel(user_intent, coords_3d, coords_2d, coords_1d, W1, b1, W2, b2, Wr,
           br, Wt, bt, Wg1, bg1, Wg2, bg2, Wgs, bgs, Wgc, bgc, Wo1, bo1, Wo2,
           bo2, Wo3, bo3, Wa1, ba1, Wa2, ba2, Wa3, ba3):
    del Wgc, bgc
    B, N, T = _B, _N, _T
    nt = N // T
    row = lambda v: v.reshape(1, -1)

    in_specs = [
        _full((B, 64)),
        pl.BlockSpec((1, T, 3), lambda b, n: (b, n, 0)),
        pl.BlockSpec((1, T, 2), lambda b, n: (b, n, 0)),
        pl.BlockSpec((1, T, 1), lambda b, n: (b, n, 0)),
        _full((64, 64)), _full((1, 64)), _full((64, 64)), _full((1, 64)),
        _full((64, 3)), _full((1, 3)), _full((64, 16)), _full((1, 16)),
        _full((19, 256)), _full((1, 256)), _full((256, 256)), _full((1, 256)),
        _full((256, 1)), _full((1, 1)),
        _full((18, 256)), _full((1, 256)), _full((256, 256)), _full((1, 256)),
        _full((256, 3)), _full((1, 3)),
        _full((17, 256)), _full((1, 256)), _full((256, 256)), _full((1, 256)),
        _full((256, 1)), _full((1, 1)),
    ]
    out_specs = [
        _full((B, 3)),
        pl.BlockSpec((1, T, 1), lambda b, n: (b, n, 0)),
        pl.BlockSpec((1, T, 3), lambda b, n: (b, n, 0)),
        pl.BlockSpec((1, T, 1), lambda b, n: (b, n, 0)),
    ]
    out_shapes = [
        jax.ShapeDtypeStruct((B, 3), jnp.float32),
        jax.ShapeDtypeStruct((B, N, 1), jnp.float32),
        jax.ShapeDtypeStruct((B, N, 3), jnp.float32),
        jax.ShapeDtypeStruct((B, N, 1), jnp.float32),
    ]

    weights, raw_sdf, raw_img, raw_audio = pl.pallas_call(
        _body,
        grid=(B, nt),
        in_specs=in_specs,
        out_specs=out_specs,
        out_shape=out_shapes,
        scratch_shapes=[
            pltpu.VMEM((B, 256), jnp.float32),
            pltpu.VMEM((B, 256), jnp.float32),
            pltpu.VMEM((B, 256), jnp.float32),
        ],
    )(user_intent, coords_3d, coords_2d, coords_1d,
      W1, row(b1), W2, row(b2), Wr, row(br), Wt, row(bt),
      Wg1, row(bg1), Wg2, row(bg2), Wgs, row(bgs),
      Wo1, row(bo1), Wo2, row(bo2), Wo3, row(bo3),
      Wa1, row(ba1), Wa2, row(ba2), Wa3, row(ba3))
    return weights, raw_sdf, raw_img, raw_audio
